# Initial kernel scaffold; baseline (speedup 1.0000x reference)
import jax
import jax.numpy as jnp
from jax.experimental import pallas as pl


def kernel(loc_data, conf_data, anchors, targets):
    raise NotImplementedError("write your pallas kernel here")


# speedup vs baseline: 48.8206x; 48.8206x over previous
import functools

import jax
import jax.numpy as jnp
from jax.experimental import pallas as pl
from jax.experimental.pallas import tpu as pltpu

_NP_RATIO = 3
_THRESHOLD = 0.5
_VAR0, _VAR1 = 0.1, 0.2


def _smooth_l1(x):
    ax = jnp.abs(x)
    return jnp.where(ax < 1.0, 0.5 * x * x, ax - 0.5)


def _loss_kernel(tgt_ref, anc_ref, loc_ref, conf_ref, out_l_ref, out_c_ref,
                 ce_ref, np_ref, *, n_frames, n_anchors, n_objs):
    bf = pl.program_id(0)
    A = n_anchors
    O = n_objs

    @pl.when(bf == 0)
    def _init():
        out_l_ref[:, :] = jnp.zeros((1, 1), jnp.float32)
        out_c_ref[:, :] = jnp.zeros((1, 1), jnp.float32)

    anc = anc_ref[:, :]
    cx, cy, w, h = anc[0:1, :], anc[1:2, :], anc[2:3, :], anc[3:4, :]
    ax1, ay1 = cx - w * 0.5, cy - h * 0.5
    ax2, ay2 = cx + w * 0.5, cy + h * 0.5
    area_a = w * h

    tgt = tgt_ref[0]
    tx1, ty1 = tgt[:, 0:1], tgt[:, 1:2]
    tx2, ty2 = tgt[:, 2:3], tgt[:, 3:4]
    area_t = (tx2 - tx1) * (ty2 - ty1)

    iw = jnp.minimum(tx2, ax2) - jnp.maximum(tx1, ax1)
    ih = jnp.minimum(ty2, ay2) - jnp.maximum(ty1, ay1)
    inter = jnp.maximum(iw, 0.0) * jnp.maximum(ih, 0.0)
    ov = inter / (area_t + area_a - inter)

    o_iota = jax.lax.broadcasted_iota(jnp.int32, (O, 1), 0)
    a_iota = jax.lax.broadcasted_iota(jnp.int32, (1, A), 1)

    bto = jnp.max(ov, axis=0, keepdims=True)
    bti = jnp.min(jnp.where(ov == bto, o_iota, O), axis=0, keepdims=True)

    m_t = jnp.max(ov, axis=1, keepdims=True)
    bpi = jnp.min(jnp.where(ov == m_t, a_iota, A), axis=1, keepdims=True)
    forced = bpi == a_iota
    f_idx = jnp.max(jnp.where(forced, o_iota, -1), axis=0, keepdims=True)
    is_f = f_idx >= 0
    bto = jnp.where(is_f, 2.0, bto)
    bti = jnp.where(is_f, f_idx, bti)

    sel_t = bti == o_iota
    mx1 = jnp.sum(jnp.where(sel_t, tx1, 0.0), axis=0, keepdims=True)
    my1 = jnp.sum(jnp.where(sel_t, ty1, 0.0), axis=0, keepdims=True)
    mx2 = jnp.sum(jnp.where(sel_t, tx2, 0.0), axis=0, keepdims=True)
    my2 = jnp.sum(jnp.where(sel_t, ty2, 0.0), axis=0, keepdims=True)

    pos = jnp.logical_not(bto < _THRESHOLD)

    g0 = ((mx1 + mx2) * 0.5 - cx) / (_VAR0 * w)
    g1 = ((my1 + my2) * 0.5 - cy) / (_VAR0 * h)
    g2 = jnp.log((mx2 - mx1) / w) / _VAR1
    g3 = jnp.log((my2 - my1) / h) / _VAR1

    loc = loc_ref[0]
    sl = (_smooth_l1(loc[0:1, :] - g0) + _smooth_l1(loc[1:2, :] - g1) +
          _smooth_l1(loc[2:3, :] - g2) + _smooth_l1(loc[3:4, :] - g3))
    out_l_ref[:, :] += jnp.sum(jnp.where(pos, sl, 0.0), axis=1, keepdims=True)

    conf = conf_ref[0]
    c0, c1 = conf[0:1, :], conf[1:2, :]
    lse = jnp.maximum(c0, c1) + jnp.log(1.0 + jnp.exp(-jnp.abs(c0 - c1)))
    ce = lse - jnp.where(pos, c1, c0)
    out_c_ref[:, :] += jnp.sum(jnp.where(pos, ce, 0.0), axis=1, keepdims=True)

    ce_ref[pl.ds(bf, 1), :] = jnp.where(pos, 0.0, ce)
    n_pos = jnp.sum(pos.astype(jnp.int32), axis=1, keepdims=True)
    np_ref[pl.ds(bf, 1), :] = jnp.broadcast_to(n_pos, (1, 128))

    @pl.when(bf == n_frames - 1)
    def _tail():
        npos = np_ref[:, 0:1]
        K = jnp.minimum(npos * _NP_RATIO, A - 1)

        def body(i, t):
            bit = jax.lax.shift_left(jnp.int32(1), jnp.int32(30) - i)
            cand = t + bit
            bits = jax.lax.bitcast_convert_type(ce_ref[:, :], jnp.int32)
            cnt = jnp.sum((bits >= cand).astype(jnp.int32), axis=1,
                          keepdims=True)
            return jnp.where(cnt >= K, cand, t)

        t0 = jnp.zeros((n_frames, 1), jnp.int32)
        t = jax.lax.fori_loop(0, 31, body, t0)
        tf = jax.lax.bitcast_convert_type(t, jnp.float32)
        V = ce_ref[:, :]
        gt = V > tf
        cnt_gt = jnp.sum(jnp.where(gt, 1.0, 0.0), axis=1, keepdims=True)
        sum_gt = jnp.sum(jnp.where(gt, V, 0.0), axis=1, keepdims=True)
        top = sum_gt + (K.astype(jnp.float32) - cnt_gt) * tf
        top = jnp.where(K > 0, top, 0.0)
        out_c_ref[:, :] += jnp.sum(top, axis=0, keepdims=True)


def kernel(loc_data, conf_data, anchors, targets):
    B = targets.shape[0]
    F = targets.shape[1]
    O = targets.shape[2]
    A = anchors.shape[0]
    BF = B * F

    loc_p = loc_data.reshape(BF, A, 4).transpose(0, 2, 1)
    conf_p = conf_data.reshape(BF, A, 2).transpose(0, 2, 1)
    tgt = targets.reshape(BF, O, 5)
    anc_t = anchors.T

    out_l, out_c = pl.pallas_call(
        functools.partial(_loss_kernel, n_frames=BF, n_anchors=A, n_objs=O),
        grid=(BF,),
        in_specs=[
            pl.BlockSpec((1, O, 5), lambda i: (i, 0, 0)),
            pl.BlockSpec((4, A), lambda i: (0, 0)),
            pl.BlockSpec((1, 4, A), lambda i: (i, 0, 0)),
            pl.BlockSpec((1, 2, A), lambda i: (i, 0, 0)),
        ],
        out_specs=[
            pl.BlockSpec((1, 1), lambda i: (0, 0)),
            pl.BlockSpec((1, 1), lambda i: (0, 0)),
        ],
        out_shape=[
            jax.ShapeDtypeStruct((1, 1), jnp.float32),
            jax.ShapeDtypeStruct((1, 1), jnp.float32),
        ],
        scratch_shapes=[
            pltpu.VMEM((BF, A), jnp.float32),
            pltpu.VMEM((BF, 128), jnp.int32),
        ],
    )(tgt, anc_t, loc_p, conf_p)
    return (out_l[0, 0], out_c[0, 0])
